# baseline (speedup 1.0000x reference)
import functools

import jax
import jax.numpy as jnp
from jax import lax
from jax.experimental import pallas as pl
from jax.experimental.pallas import tpu as pltpu
from jax.experimental.pallas import tpu_sc as plsc

N_USERS = 50000
N_ITEMS = 50000
N = N_USERS + N_ITEMS
E = 1600000
N_LAYERS = 3
D = 32

NC = 2
NS = 16
QUART = N // 4
CHUNK = 1024
MICRO = 128
NMICRO = CHUNK // MICRO
NCHUNK = 98
E_PAD = NS * NCHUNK * CHUNK
SPAD_PER_TILE = 1664
SPAD_ROWS = NS * SPAD_PER_TILE
DUMMY = QUART
OUT_PER_TILE = 1568

NW = NC * NS
BCHUNK = 49
CAP = BCHUNK * CHUNK + CHUNK
NREG = NW * 4
BTOT = NREG * CAP


def _bucket_body(rows_hbm, cols_hbm, vals_hbm,
                 blr_hbm, bcl_hbm, bvl_hbm, cnt_hbm,
                 rows_f, cols_f, vals_f, lrow_f, dest2f,
                 padl, padc, padv, cbuf, sem, sem2):
    c = lax.axis_index("c")
    s = lax.axis_index("s")
    wid = c * NS + s
    eye16 = lax.iota(jnp.int32, 16)
    gbs = [(wid * 4 + q) * CAP for q in range(4)]

    for l in range(8):
        padl[pl.ds(l * 16, 16)] = jnp.full((16,), DUMMY, jnp.int32)
        padc[pl.ds(l * 16, 16)] = jnp.zeros((16,), jnp.int32)
        padv[pl.ds(l * 16, 16)] = jnp.zeros((16,), jnp.float32)

    def chunk_body(ch, woff):
        ebase = (wid * BCHUNK + ch) * CHUNK
        h = [pltpu.async_copy(rows_hbm.at[pl.ds(ebase, CHUNK)], rows_f, sem2),
             pltpu.async_copy(cols_hbm.at[pl.ds(ebase, CHUNK)], cols_f, sem2),
             pltpu.async_copy(vals_hbm.at[pl.ds(ebase, CHUNK)], vals_f, sem2)]
        for x in h:
            x.wait()

        def group_body(g, w4):
            w0, w1, w2, w3 = w4
            b = g * 16
            r = rows_f[pl.ds(b, 16)]
            qv = (jnp.where(r >= QUART, 1, 0)
                  + jnp.where(r >= 2 * QUART, 1, 0)
                  + jnp.where(r >= 3 * QUART, 1, 0))
            lrow_f[pl.ds(b, 16)] = r - qv * QUART
            base = jnp.where(qv == 0, gbs[0] + w0,
                             jnp.where(qv == 1, gbs[1] + w1,
                                       jnp.where(qv == 2, gbs[2] + w2,
                                                 gbs[3] + w3)))
            qks = [qv[j] for j in range(16)]
            rank = jnp.zeros((16,), jnp.int32)
            for j in range(15):
                mm = (qv == qks[j]) & (eye16 > j)
                rank = rank + jnp.where(mm, 1, 0)
            dest2f[pl.ds(b, 16)] = base + rank
            one = jnp.int32(1)
            zero = jnp.int32(0)
            t0 = sum([jnp.where(qk == 0, one, zero) for qk in qks])
            t1 = sum([jnp.where(qk == 1, one, zero) for qk in qks])
            t2 = sum([jnp.where(qk == 2, one, zero) for qk in qks])
            t3 = sum([jnp.where(qk == 3, one, zero) for qk in qks])
            return (w0 + t0, w1 + t1, w2 + t2, w3 + t3)
        woff = lax.fori_loop(0, CHUNK // 16, group_body, woff)

        hs = []
        for j in range(NMICRO):
            sl = pl.ds(j * MICRO, MICRO)
            dj = dest2f.at[pl.ds(j * MICRO, MICRO)]
            hs.append(pltpu.async_copy(lrow_f.at[sl], blr_hbm.at[dj], sem))
            hs.append(pltpu.async_copy(cols_f.at[sl], bcl_hbm.at[dj], sem))
            hs.append(pltpu.async_copy(vals_f.at[sl], bvl_hbm.at[dj], sem))
        for x in hs:
            x.wait()
        return woff

    z = jnp.int32(0)
    woff = lax.fori_loop(0, BCHUNK, chunk_body, (z, z, z, z))

    ms = []
    hs = []
    for q in range(4):
        w = woff[q]
        for j in range(NMICRO):
            for l in range(8):
                dest2f[pl.ds(j * MICRO + l * 16, 16)] = (
                    gbs[q] + w + j * MICRO + l * 16 + eye16)
        for j in range(NMICRO):
            dj = dest2f.at[pl.ds(j * MICRO, MICRO)]
            hs.append(pltpu.async_copy(padl, blr_hbm.at[dj], sem))
            hs.append(pltpu.async_copy(padc, bcl_hbm.at[dj], sem))
            hs.append(pltpu.async_copy(padv, bvl_hbm.at[dj], sem))
        for x in hs:
            x.wait()
        hs = []
        ms.append((w + CHUNK - 1) // CHUNK)

    cvec = (jnp.where(eye16 == 0, ms[0], 0)
            + jnp.where(eye16 == 1, ms[1], 0)
            + jnp.where(eye16 == 2, ms[2], 0)
            + jnp.where(eye16 == 3, ms[3], 0))
    cbuf[pl.ds(0, 16)] = cvec
    pltpu.sync_copy(cbuf, cnt_hbm.at[pl.ds(wid * 16, 16)])


_bucket = pl.kernel(
    _bucket_body,
    out_type=[jax.ShapeDtypeStruct((BTOT,), jnp.int32),
              jax.ShapeDtypeStruct((BTOT,), jnp.int32),
              jax.ShapeDtypeStruct((BTOT,), jnp.float32),
              jax.ShapeDtypeStruct((NW * 16,), jnp.int32)],
    mesh=plsc.VectorSubcoreMesh(core_axis_name="c", subcore_axis_name="s"),
    scratch_types=[
        pltpu.VMEM((CHUNK,), jnp.int32),
        pltpu.VMEM((CHUNK,), jnp.int32),
        pltpu.VMEM((CHUNK,), jnp.float32),
        pltpu.VMEM((CHUNK,), jnp.int32),
        pltpu.VMEM((CHUNK,), jnp.int32),
        pltpu.VMEM((MICRO,), jnp.int32),
        pltpu.VMEM((MICRO,), jnp.int32),
        pltpu.VMEM((MICRO,), jnp.float32),
        pltpu.VMEM((16,), jnp.int32),
        pltpu.SemaphoreType.DMA,
        pltpu.SemaphoreType.DMA,
    ],
    compiler_params=pltpu.CompilerParams(use_tc_tiling_on_sc=False),
)


def _spmm_body(bcl_hbm, blr_hbm, bvl_hbm, cnt_hbm, ego_hbm, out_hbm,
               cols2, idx2, cbuf, vals_f, gath, zbuf, sem, sem2, sem3, spad):
    c = lax.axis_index("c")
    s = lax.axis_index("s")

    zv = jnp.zeros((16,), jnp.float32)
    for r in range(MICRO):
        zbuf[r, pl.ds(0, 16)] = zv
        zbuf[r, pl.ds(16, 16)] = zv

    for p in range(2):
        base = (c * 2 + p) * QUART

        def zero_body(k, _):
            pltpu.sync_copy(zbuf,
                            spad.at[pl.ds(s * SPAD_PER_TILE + k * MICRO,
                                          MICRO)])
            return 0
        lax.fori_loop(0, SPAD_PER_TILE // MICRO, zero_body, 0)
        plsc.subcore_barrier()

        for t_off in range(2):
            t = s * 2 + t_off
            pltpu.sync_copy(cnt_hbm.at[pl.ds(t * 16, 16)], cbuf)
            cv = cbuf[pl.ds(0, 16)]
            nch = jnp.where(c == 0, cv[p], cv[2 + p])
            gelem = (t * 4 + c * 2 + p) * CAP
            gmicro = gelem // MICRO

            def chunk_body(ch, _):
                ebase = gelem + ch * CHUNK
                mbase = gmicro + ch * NMICRO
                h = [pltpu.async_copy(bcl_hbm.at[pl.ds(mbase, NMICRO)],
                                      cols2, sem2),
                     pltpu.async_copy(blr_hbm.at[pl.ds(mbase, NMICRO)],
                                      idx2, sem2),
                     pltpu.async_copy(bvl_hbm.at[pl.ds(ebase, CHUNK)],
                                      vals_f, sem2)]
                for x in h:
                    x.wait()

                handles = []
                for j in range(NMICRO):
                    handles.append(pltpu.async_copy(
                        ego_hbm.at[cols2.at[j]],
                        gath.at[pl.ds(j * MICRO, MICRO)], sem))
                for x in handles:
                    x.wait()

                def scale_body(grp, _):
                    b = grp * 16
                    vv = vals_f[pl.ds(b, 16)]
                    for k in range(16):
                        e = b + k
                        v = vv[k]
                        gath[e, pl.ds(0, 16)] = gath[e, pl.ds(0, 16)] * v
                        gath[e, pl.ds(16, 16)] = gath[e, pl.ds(16, 16)] * v
                    return 0
                lax.fori_loop(0, CHUNK // 16, scale_body, 0)

                hs = []
                for j in range(NMICRO):
                    hs.append(pltpu.async_copy(
                        gath.at[pl.ds(j * MICRO, MICRO)],
                        spad.at[idx2.at[j]], sem3, add=True))
                for x in hs:
                    x.wait()
                return 0
            lax.fori_loop(0, nch, chunk_body, 0)
        plsc.subcore_barrier()

        r0 = s * OUT_PER_TILE
        o0 = base + r0

        def out_body(k, _):
            pltpu.sync_copy(spad.at[pl.ds(r0 + k * MICRO, MICRO)],
                            gath.at[pl.ds(0, MICRO)])
            pltpu.sync_copy(gath.at[pl.ds(0, MICRO)],
                            out_hbm.at[pl.ds(o0 + k * MICRO, MICRO)])
            return 0
        nfull = jnp.where(s < NS - 1, 12, 11)
        lax.fori_loop(0, nfull, out_body, 0)

        @pl.when(s < NS - 1)
        def _():
            pltpu.sync_copy(spad.at[pl.ds(r0 + 12 * MICRO, 32)],
                            gath.at[pl.ds(0, 32)])
            pltpu.sync_copy(gath.at[pl.ds(0, 32)],
                            out_hbm.at[pl.ds(o0 + 12 * MICRO, 32)])

        @pl.when(s == NS - 1)
        def _():
            pltpu.sync_copy(spad.at[pl.ds(r0 + 11 * MICRO, 72)],
                            gath.at[pl.ds(0, 72)])
            pltpu.sync_copy(gath.at[pl.ds(0, 72)],
                            out_hbm.at[pl.ds(o0 + 11 * MICRO, 72)])
        plsc.subcore_barrier()


_spmm = pl.kernel(
    _spmm_body,
    out_type=jax.ShapeDtypeStruct((N, D), jnp.float32),
    mesh=plsc.VectorSubcoreMesh(core_axis_name="c", subcore_axis_name="s"),
    scratch_types=[
        pltpu.VMEM((NMICRO, MICRO), jnp.int32),
        pltpu.VMEM((NMICRO, MICRO), jnp.int32),
        pltpu.VMEM((16,), jnp.int32),
        pltpu.VMEM((CHUNK,), jnp.float32),
        pltpu.VMEM((CHUNK, D), jnp.float32),
        pltpu.VMEM((MICRO, D), jnp.float32),
        pltpu.SemaphoreType.DMA,
        pltpu.SemaphoreType.DMA,
        pltpu.SemaphoreType.DMA,
        pltpu.VMEM_SHARED((SPAD_ROWS, D), jnp.float32),
    ],
    compiler_params=pltpu.CompilerParams(use_tc_tiling_on_sc=False),
)


def _enc_body(text_ref, w1_ref, w2_ref, id_ref, out_ref):
    h = jnp.maximum(jnp.dot(text_ref[...], w1_ref[...],
                            preferred_element_type=jnp.float32), 0.0)
    created = jnp.dot(h, w2_ref[...], preferred_element_type=jnp.float32)
    out_ref[...] = jnp.concatenate([id_ref[...], created], axis=1)


def _encode(text, w1, w2, id_emb, bm=400):
    n = text.shape[0]
    grid = n // bm
    return pl.pallas_call(
        _enc_body,
        grid=(grid,),
        in_specs=[
            pl.BlockSpec((bm, text.shape[1]), lambda i: (i, 0)),
            pl.BlockSpec(w1.shape, lambda i: (0, 0)),
            pl.BlockSpec(w2.shape, lambda i: (0, 0)),
            pl.BlockSpec((bm, id_emb.shape[1]), lambda i: (i, 0)),
        ],
        out_specs=pl.BlockSpec((bm, D), lambda i: (i, 0)),
        out_shape=jax.ShapeDtypeStruct((n, D), jnp.float32),
    )(text, w1, w2, id_emb)


def _leaky(x):
    return jnp.where(x >= 0, x, 0.01 * x)


def _dense_body(side_ref, ego_ref, gcw_ref, gcb_ref, biw_ref, bib_ref,
                new_ref, nrm_ref):
    sde = side_ref[...]
    ego = ego_ref[...]
    sum_e = _leaky(jnp.dot(sde, gcw_ref[...],
                           preferred_element_type=jnp.float32) + gcb_ref[...])
    bi = _leaky(jnp.dot(ego * sde, biw_ref[...],
                        preferred_element_type=jnp.float32) + bib_ref[...])
    ne = sum_e + bi
    nrm = jnp.sqrt(jnp.sum(ne * ne, axis=1, keepdims=True))
    new_ref[...] = ne
    nrm_ref[...] = ne / jnp.maximum(nrm, 1e-12)


def _dense_update(side, ego, gcw, gcb, biw, bib, bm=1000):
    grid = N // bm
    full = lambda a: pl.BlockSpec(a.shape, lambda i: (0, 0))
    return pl.pallas_call(
        _dense_body,
        grid=(grid,),
        in_specs=[
            pl.BlockSpec((bm, D), lambda i: (i, 0)),
            pl.BlockSpec((bm, D), lambda i: (i, 0)),
            full(gcw), full(gcb), full(biw), full(bib),
        ],
        out_specs=[pl.BlockSpec((bm, D), lambda i: (i, 0)),
                   pl.BlockSpec((bm, D), lambda i: (i, 0))],
        out_shape=[jax.ShapeDtypeStruct((N, D), jnp.float32),
                   jax.ShapeDtypeStruct((N, D), jnp.float32)],
    )(side, ego, gcw, gcb, biw, bib)


def kernel(adj_indices, adj_values, user_name_embs, sent_embs, user_emb,
           item_emb, u_w1, u_w2, i_w1, i_w2, gc_w, gc_b, bi_w, bi_b):
    ego_u = _encode(user_name_embs, u_w1, u_w2, user_emb)
    ego_i = _encode(sent_embs, i_w1, i_w2, item_emb)
    ego = jnp.concatenate([ego_u, ego_i], axis=0)

    rows = adj_indices[0].astype(jnp.int32)
    cols = adj_indices[1].astype(jnp.int32)
    pad = E_PAD - E
    rows_p = jnp.concatenate([rows, jnp.full((pad,), N, jnp.int32)])
    cols_p = jnp.concatenate([cols, jnp.zeros((pad,), jnp.int32)])
    vals_p = jnp.concatenate([adj_values,
                              jnp.zeros((pad,), jnp.float32)])

    blr, bcl, bvl, cnts = _bucket(rows_p, cols_p, vals_p)
    bcl2 = bcl.reshape(BTOT // MICRO, MICRO)
    blr2 = blr.reshape(BTOT // MICRO, MICRO)

    outs = [ego]
    for i in range(N_LAYERS):
        side = _spmm(bcl2, blr2, bvl, cnts, ego)
        ego, nrm = _dense_update(side, ego, gc_w[i], gc_b[i].reshape(1, D),
                                 bi_w[i], bi_b[i].reshape(1, D))
        outs.append(nrm)
    all_e = jnp.concatenate(outs, axis=1)
    return all_e[:N_USERS], all_e[N_USERS:]
